# baseline (speedup 1.0000x reference)
import jax
import jax.numpy as jnp
from jax import lax
from jax.experimental import pallas as pl
from jax.experimental.pallas import tpu as pltpu, tpu_sc as plsc

_NC, _NS = 2, 16
_NW = _NC * _NS
_B, _L, _D = 8, 2048, 512
_WPB = _NW // _B
_QROWS = _L // _WPB
_NBLK = 8
_BLK = _QROWS // _NBLK
_CHUNKS = _L // 16


def _sc_body(x_hbm, at_hbm, out_hbm, at_v, perm_v, buf0, buf1, buf2,
             gsem0, gsem1, gsem2, wsem0, wsem1, wsem2):
    cid = lax.axis_index("c")
    sid = lax.axis_index("s")
    wid = sid * _NC + cid
    b = wid // _WPB
    q = wid % _WPB

    pltpu.sync_copy(at_hbm.at[b], at_v)

    lanes = jnp.arange(16, dtype=jnp.int32)
    row_base = b * _L
    ones = jnp.full((16,), 1, jnp.int32)
    zeros = jnp.full((16,), 0, jnp.int32)
    twos = jnp.full((16,), 2, jnp.int32)

    def count_body(k, nb):
        v = at_v[pl.ds(k * 16, 16)]
        m = (v == zeros) | (v == ones) | (v == twos)
        mi = jnp.where(m, ones, zeros)
        return nb + jnp.sum(mi)

    nb = lax.fori_loop(0, _CHUNKS, count_body, jnp.int32(0))

    def perm_body(k, carry):
        bbc, nbc = carry
        v = at_v[pl.ds(k * 16, 16)]
        m = (v == zeros) | (v == ones) | (v == twos)
        mi = jnp.where(m, ones, zeros)
        cs = jnp.cumsum(mi)
        csn = lanes + ones - cs
        bb_dest = jnp.full((16,), bbc - 1, jnp.int32) + cs
        nbb_dest = jnp.full((16,), nb + nbc - 1, jnp.int32) + csn
        dest = jnp.where(m, bb_dest, nbb_dest)
        src = row_base + k * 16 + lanes
        plsc.store_scatter(perm_v, [dest], src)
        pc = jnp.sum(mi)
        return (bbc + pc, nbc + (16 - pc))

    lax.fori_loop(0, _CHUNKS, perm_body, (jnp.int32(0), jnp.int32(0)))

    out_base = row_base + q * _QROWS
    idx_base = q * _QROWS
    bufs = (buf0, buf1, buf2)
    gsems = (gsem0, gsem1, gsem2)
    wsems = (wsem0, wsem1, wsem2)

    def gather(blk):
        s = blk % 3
        return pltpu.async_copy(
            x_hbm.at[perm_v.at[pl.ds(idx_base + blk * _BLK, _BLK)]],
            bufs[s], gsems[s])

    def write(blk):
        s = blk % 3
        return pltpu.async_copy(
            bufs[s], out_hbm.at[pl.ds(out_base + blk * _BLK, _BLK)],
            wsems[s])

    gd = [None] * _NBLK
    wd = [None] * _NBLK
    gd[0] = gather(0)
    for blk in range(_NBLK):
        if blk >= 2:
            wd[blk - 2].wait()
        if blk + 1 < _NBLK:
            gd[blk + 1] = gather(blk + 1)
        gd[blk].wait()
        wd[blk] = write(blk)
    wd[_NBLK - 2].wait()
    wd[_NBLK - 1].wait()

def _sc_permute(x2, at32):
    mesh = plsc.VectorSubcoreMesh(core_axis_name="c", subcore_axis_name="s")
    k = pl.kernel(
        _sc_body,
        out_type=jax.ShapeDtypeStruct((_B * _L, _D), jnp.float32),
        mesh=mesh,
        compiler_params=pltpu.CompilerParams(needs_layout_passes=False),
        scratch_types=[
            pltpu.VMEM((_L,), jnp.int32),
            pltpu.VMEM((_L,), jnp.int32),
            pltpu.VMEM((_BLK, _D), jnp.float32),
            pltpu.VMEM((_BLK, _D), jnp.float32),
            pltpu.VMEM((_BLK, _D), jnp.float32),
            pltpu.SemaphoreType.DMA,
            pltpu.SemaphoreType.DMA,
            pltpu.SemaphoreType.DMA,
            pltpu.SemaphoreType.DMA,
            pltpu.SemaphoreType.DMA,
            pltpu.SemaphoreType.DMA,
        ],
    )
    return k(x2, at32)


@jax.jit
def kernel(x, atom_type, aa_type):
    x2 = x.reshape(_B * _L, _D)
    at32 = atom_type.astype(jnp.int32)
    out = _sc_permute(x2, at32)
    return out.reshape(_B, _L, _D)


# speedup vs baseline: 1.0232x; 1.0232x over previous
import jax
import jax.numpy as jnp
from jax import lax
from jax.experimental import pallas as pl
from jax.experimental.pallas import tpu as pltpu, tpu_sc as plsc

_NC, _NS = 2, 16
_NW = _NC * _NS
_B, _L, _D = 8, 2048, 512
_WPB = _NW // _B
_QROWS = _L // _WPB
_NBLK = 16
_BLK = _QROWS // _NBLK
_CHUNKS = _L // 16


def _sc_body(x_hbm, at_hbm, out_hbm, at_v, perm_v, buf0, buf1, buf2, buf3,
             gsem0, gsem1, gsem2, gsem3, wsem0, wsem1, wsem2, wsem3):
    cid = lax.axis_index("c")
    sid = lax.axis_index("s")
    wid = sid * _NC + cid
    b = wid // _WPB
    q = wid % _WPB

    pltpu.sync_copy(at_hbm.at[b], at_v)

    lanes = jnp.arange(16, dtype=jnp.int32)
    row_base = b * _L
    ones = jnp.full((16,), 1, jnp.int32)
    zeros = jnp.full((16,), 0, jnp.int32)
    twos = jnp.full((16,), 2, jnp.int32)

    def count_body(k, nb):
        v = at_v[pl.ds(k * 16, 16)]
        m = (v == zeros) | (v == ones) | (v == twos)
        mi = jnp.where(m, ones, zeros)
        return nb + jnp.sum(mi)

    nb = lax.fori_loop(0, _CHUNKS, count_body, jnp.int32(0))

    def perm_body(k, carry):
        bbc, nbc = carry
        v = at_v[pl.ds(k * 16, 16)]
        m = (v == zeros) | (v == ones) | (v == twos)
        mi = jnp.where(m, ones, zeros)
        cs = jnp.cumsum(mi)
        csn = lanes + ones - cs
        bb_dest = jnp.full((16,), bbc - 1, jnp.int32) + cs
        nbb_dest = jnp.full((16,), nb + nbc - 1, jnp.int32) + csn
        dest = jnp.where(m, bb_dest, nbb_dest)
        src = row_base + k * 16 + lanes
        plsc.store_scatter(perm_v, [dest], src)
        pc = jnp.sum(mi)
        return (bbc + pc, nbc + (16 - pc))

    lax.fori_loop(0, _CHUNKS, perm_body, (jnp.int32(0), jnp.int32(0)))

    out_base = row_base + q * _QROWS
    idx_base = q * _QROWS
    bufs = (buf0, buf1, buf2, buf3)
    gsems = (gsem0, gsem1, gsem2, gsem3)
    wsems = (wsem0, wsem1, wsem2, wsem3)

    def gather(blk):
        s = blk % 4
        return pltpu.async_copy(
            x_hbm.at[perm_v.at[pl.ds(idx_base + blk * _BLK, _BLK)]],
            bufs[s], gsems[s])

    def write(blk):
        s = blk % 4
        return pltpu.async_copy(
            bufs[s], out_hbm.at[pl.ds(out_base + blk * _BLK, _BLK)],
            wsems[s])

    gd = [None] * _NBLK
    wd = [None] * _NBLK
    gd[0] = gather(0)
    gd[1] = gather(1)
    for blk in range(_NBLK):
        if blk >= 2:
            wd[blk - 2].wait()
        if blk + 2 < _NBLK:
            gd[blk + 2] = gather(blk + 2)
        gd[blk].wait()
        wd[blk] = write(blk)
    wd[_NBLK - 2].wait()
    wd[_NBLK - 1].wait()

def _sc_permute(x2, at32):
    mesh = plsc.VectorSubcoreMesh(core_axis_name="c", subcore_axis_name="s")
    k = pl.kernel(
        _sc_body,
        out_type=jax.ShapeDtypeStruct((_B * _L, _D), jnp.float32),
        mesh=mesh,
        compiler_params=pltpu.CompilerParams(needs_layout_passes=False),
        scratch_types=[
            pltpu.VMEM((_L,), jnp.int32),
            pltpu.VMEM((_L,), jnp.int32),
            pltpu.VMEM((_BLK, _D), jnp.float32),
            pltpu.VMEM((_BLK, _D), jnp.float32),
            pltpu.VMEM((_BLK, _D), jnp.float32),
            pltpu.VMEM((_BLK, _D), jnp.float32),
            pltpu.SemaphoreType.DMA,
            pltpu.SemaphoreType.DMA,
            pltpu.SemaphoreType.DMA,
            pltpu.SemaphoreType.DMA,
            pltpu.SemaphoreType.DMA,
            pltpu.SemaphoreType.DMA,
            pltpu.SemaphoreType.DMA,
            pltpu.SemaphoreType.DMA,
        ],
    )
    return k(x2, at32)


@jax.jit
def kernel(x, atom_type, aa_type):
    x2 = x.reshape(_B * _L, _D)
    at32 = atom_type.astype(jnp.int32)
    out = _sc_permute(x2, at32)
    return out.reshape(_B, _L, _D)


# speedup vs baseline: 1.0395x; 1.0159x over previous
import jax
import jax.numpy as jnp
from jax import lax
from jax.experimental import pallas as pl
from jax.experimental.pallas import tpu as pltpu, tpu_sc as plsc

_NC, _NS = 2, 16
_NW = _NC * _NS
_B, _L, _D = 8, 2048, 512
_WPB = _NW // _B
_QROWS = _L // _WPB
_NBLK = 16
_BLK = _QROWS // _NBLK
_CHUNKS = _L // 16


def _sc_body(x_hbm, at_hbm, out_hbm, at_v, perm_v,
             buf0, buf1, buf2, buf3, buf4, buf5,
             gsem0, gsem1, gsem2, gsem3, gsem4, gsem5,
             wsem0, wsem1, wsem2, wsem3, wsem4, wsem5):
    cid = lax.axis_index("c")
    sid = lax.axis_index("s")
    wid = sid * _NC + cid
    b = wid // _WPB
    q = wid % _WPB

    pltpu.sync_copy(at_hbm.at[b], at_v)

    lanes = jnp.arange(16, dtype=jnp.int32)
    row_base = b * _L
    ones = jnp.full((16,), 1, jnp.int32)
    zeros = jnp.full((16,), 0, jnp.int32)
    twos = jnp.full((16,), 2, jnp.int32)

    def count_body(k, nb):
        v = at_v[pl.ds(k * 16, 16)]
        m = (v == zeros) | (v == ones) | (v == twos)
        mi = jnp.where(m, ones, zeros)
        return nb + jnp.sum(mi)

    nb = lax.fori_loop(0, _CHUNKS, count_body, jnp.int32(0))

    def perm_body(k, carry):
        bbc, nbc = carry
        v = at_v[pl.ds(k * 16, 16)]
        m = (v == zeros) | (v == ones) | (v == twos)
        mi = jnp.where(m, ones, zeros)
        cs = jnp.cumsum(mi)
        csn = lanes + ones - cs
        bb_dest = jnp.full((16,), bbc - 1, jnp.int32) + cs
        nbb_dest = jnp.full((16,), nb + nbc - 1, jnp.int32) + csn
        dest = jnp.where(m, bb_dest, nbb_dest)
        src = row_base + k * 16 + lanes
        plsc.store_scatter(perm_v, [dest], src)
        pc = jnp.sum(mi)
        return (bbc + pc, nbc + (16 - pc))

    lax.fori_loop(0, _CHUNKS, perm_body, (jnp.int32(0), jnp.int32(0)))

    out_base = row_base + q * _QROWS
    idx_base = q * _QROWS
    bufs = (buf0, buf1, buf2, buf3, buf4, buf5)
    gsems = (gsem0, gsem1, gsem2, gsem3, gsem4, gsem5)
    wsems = (wsem0, wsem1, wsem2, wsem3, wsem4, wsem5)

    def gather(blk):
        s = blk % 6
        return pltpu.async_copy(
            x_hbm.at[perm_v.at[pl.ds(idx_base + blk * _BLK, _BLK)]],
            bufs[s], gsems[s])

    def write(blk):
        s = blk % 6
        return pltpu.async_copy(
            bufs[s], out_hbm.at[pl.ds(out_base + blk * _BLK, _BLK)],
            wsems[s])

    gd = [None] * _NBLK
    wd = [None] * _NBLK
    gd[0] = gather(0)
    gd[1] = gather(1)
    gd[2] = gather(2)
    gd[3] = gather(3)
    for blk in range(_NBLK):
        if blk >= 2:
            wd[blk - 2].wait()
        if blk + 4 < _NBLK:
            gd[blk + 4] = gather(blk + 4)
        gd[blk].wait()
        wd[blk] = write(blk)
    wd[_NBLK - 2].wait()
    wd[_NBLK - 1].wait()

def _sc_permute(x2, at32):
    mesh = plsc.VectorSubcoreMesh(core_axis_name="c", subcore_axis_name="s")
    k = pl.kernel(
        _sc_body,
        out_type=jax.ShapeDtypeStruct((_B * _L, _D), jnp.float32),
        mesh=mesh,
        compiler_params=pltpu.CompilerParams(needs_layout_passes=False),
        scratch_types=[
            pltpu.VMEM((_L,), jnp.int32),
            pltpu.VMEM((_L,), jnp.int32),
            pltpu.VMEM((_BLK, _D), jnp.float32),
            pltpu.VMEM((_BLK, _D), jnp.float32),
            pltpu.VMEM((_BLK, _D), jnp.float32),
            pltpu.VMEM((_BLK, _D), jnp.float32),
            pltpu.VMEM((_BLK, _D), jnp.float32),
            pltpu.VMEM((_BLK, _D), jnp.float32),
            pltpu.SemaphoreType.DMA,
            pltpu.SemaphoreType.DMA,
            pltpu.SemaphoreType.DMA,
            pltpu.SemaphoreType.DMA,
            pltpu.SemaphoreType.DMA,
            pltpu.SemaphoreType.DMA,
            pltpu.SemaphoreType.DMA,
            pltpu.SemaphoreType.DMA,
            pltpu.SemaphoreType.DMA,
            pltpu.SemaphoreType.DMA,
            pltpu.SemaphoreType.DMA,
            pltpu.SemaphoreType.DMA,

        ],
    )
    return k(x2, at32)


@jax.jit
def kernel(x, atom_type, aa_type):
    x2 = x.reshape(_B * _L, _D)
    at32 = atom_type.astype(jnp.int32)
    out = _sc_permute(x2, at32)
    return out.reshape(_B, _L, _D)


# speedup vs baseline: 1.3339x; 1.2832x over previous
import jax
import jax.numpy as jnp
from jax import lax
from jax.experimental import pallas as pl
from jax.experimental.pallas import tpu as pltpu, tpu_sc as plsc

_NC, _NS = 2, 16
_NW = _NC * _NS
_B, _L, _D = 8, 2048, 512
_WPB = _NW // _B
_QROWS = _L // _WPB
_NBLK = 16
_BLK = _QROWS // _NBLK
_CHUNKS = _L // 16


def _sc_body(x_hbm, at_hbm, out_hbm, at_v, perm_v,
             buf0, buf1, buf2, buf3, buf4, buf5,
             gsem0, gsem1, gsem2, gsem3, gsem4, gsem5,
             wsem0, wsem1, wsem2, wsem3, wsem4, wsem5):
    cid = lax.axis_index("c")
    sid = lax.axis_index("s")
    wid = sid * _NC + cid
    b = wid // _WPB
    q = wid % _WPB

    pltpu.sync_copy(at_hbm.at[b], at_v)

    lanes = jnp.arange(16, dtype=jnp.int32)
    row_base = b * _L
    ones = jnp.full((16,), 1, jnp.int32)
    zeros = jnp.full((16,), 0, jnp.int32)
    twos = jnp.full((16,), 2, jnp.int32)

    def count_body(k, nb):
        v = at_v[pl.ds(k * 16, 16)]
        m = (v == zeros) | (v == ones) | (v == twos)
        mi = jnp.where(m, ones, zeros)
        return nb + jnp.sum(mi)

    nb = lax.fori_loop(0, _CHUNKS, count_body, jnp.int32(0))

    def perm_body(k, carry):
        bbc, nbc = carry
        v = at_v[pl.ds(k * 16, 16)]
        m = (v == zeros) | (v == ones) | (v == twos)
        mi = jnp.where(m, ones, zeros)
        cs = jnp.cumsum(mi)
        csn = lanes + ones - cs
        bb_dest = jnp.full((16,), bbc - 1, jnp.int32) + cs
        nbb_dest = jnp.full((16,), nb + nbc - 1, jnp.int32) + csn
        dest = jnp.where(m, bb_dest, nbb_dest)
        src = row_base + k * 16 + lanes
        plsc.store_scatter(perm_v, [dest], src)
        pc = jnp.sum(mi)
        return (bbc + pc, nbc + (16 - pc))

    lax.fori_loop(0, _CHUNKS, perm_body, (jnp.int32(0), jnp.int32(0)))

    out_base = row_base + q * _QROWS
    idx_base = q * _QROWS
    bufs = (buf0, buf1, buf2, buf3, buf4, buf5)
    gsems = (gsem0, gsem1, gsem2, gsem3, gsem4, gsem5)
    wsems = (wsem0, wsem1, wsem2, wsem3, wsem4, wsem5)

    def gather(blk):
        s = blk % 6
        return pltpu.async_copy(
            x_hbm.at[perm_v.at[pl.ds(idx_base + blk * _BLK, _BLK)]],
            bufs[s], gsems[s])

    def write(blk):
        s = blk % 6
        return pltpu.async_copy(
            bufs[s], out_hbm.at[pl.ds(out_base + blk * _BLK, _BLK)],
            wsems[s])

    gd = [None] * _NBLK
    for blk in range(_NBLK):
        gd[blk] = gather(blk)
        if blk >= 5:
            gd[blk - 5].wait()
    for blk in range(_NBLK - 5, _NBLK):
        gd[blk].wait()

def _sc_permute(x2, at32):
    mesh = plsc.VectorSubcoreMesh(core_axis_name="c", subcore_axis_name="s")
    k = pl.kernel(
        _sc_body,
        out_type=jax.ShapeDtypeStruct((_B * _L, _D), jnp.float32),
        mesh=mesh,
        compiler_params=pltpu.CompilerParams(needs_layout_passes=False),
        scratch_types=[
            pltpu.VMEM((_L,), jnp.int32),
            pltpu.VMEM((_L,), jnp.int32),
            pltpu.VMEM((_BLK, _D), jnp.float32),
            pltpu.VMEM((_BLK, _D), jnp.float32),
            pltpu.VMEM((_BLK, _D), jnp.float32),
            pltpu.VMEM((_BLK, _D), jnp.float32),
            pltpu.VMEM((_BLK, _D), jnp.float32),
            pltpu.VMEM((_BLK, _D), jnp.float32),
            pltpu.SemaphoreType.DMA,
            pltpu.SemaphoreType.DMA,
            pltpu.SemaphoreType.DMA,
            pltpu.SemaphoreType.DMA,
            pltpu.SemaphoreType.DMA,
            pltpu.SemaphoreType.DMA,
            pltpu.SemaphoreType.DMA,
            pltpu.SemaphoreType.DMA,
            pltpu.SemaphoreType.DMA,
            pltpu.SemaphoreType.DMA,
            pltpu.SemaphoreType.DMA,
            pltpu.SemaphoreType.DMA,

        ],
    )
    return k(x2, at32)


@jax.jit
def kernel(x, atom_type, aa_type):
    x2 = x.reshape(_B * _L, _D)
    at32 = atom_type.astype(jnp.int32)
    out = _sc_permute(x2, at32)
    return out.reshape(_B, _L, _D)
